# baseline (speedup 1.0000x reference)
import functools

import jax
import jax.numpy as jnp
from jax import lax
from jax.experimental import pallas as pl
from jax.experimental.pallas import tpu as pltpu
from jax.experimental.pallas import tpu_sc as plsc

N_NODES = 100000
N_EDGES = 6400000

NC, NS = 2, 16
NW = NC * NS
EPW = 199680
CE = 2560
NCH = EPW // CE
NBUF = 3
GU = 8
EXTRA0 = NW * EPW
EXTRA_CE = 1024
N_EXTRA = (N_EDGES - EXTRA0) // EXTRA_CE

SLICE = 6256
ACC_PAD = NS * SLICE
PIECES = (2560, 2560, 1136)


def _sc_body(x_hbm, edge_hbm, zeros_hbm, out_hbm,
             src_v0, src_v1, src_v2, dst_v0, dst_v1, dst_v2,
             val_v0, val_v1, val_v2, x_v, acc_sh,
             lsem0, lsem1, lsem2, ssem0, ssem1, ssem2, xsem):
    src_v = (src_v0, src_v1, src_v2)
    dst_v = (dst_v0, dst_v1, dst_v2)
    val_v = (val_v0, val_v1, val_v2)
    lsem = (lsem0, lsem1, lsem2)
    ssem = (ssem0, ssem1, ssem2)
    c = lax.axis_index("c")
    s = lax.axis_index("s")
    wid = s * NC + c

    xcp = pltpu.async_copy(x_hbm, x_v, xsem)
    off = 0
    for piece in PIECES:
        pltpu.sync_copy(zeros_hbm.at[pl.ds(0, piece)],
                        val_v0.at[pl.ds(0, piece)])
        pltpu.sync_copy(val_v0.at[pl.ds(0, piece)],
                        acc_sh.at[pl.ds(s * SLICE + off, piece)])
        off += piece

    base_e = wid * EPW

    def load_idx(ic):
        b = ic % NBUF
        e0 = base_e + ic * CE
        return [pltpu.async_copy(edge_hbm.at[0, pl.ds(e0, CE)], src_v[b],
                                 lsem[b]),
                pltpu.async_copy(edge_hbm.at[1, pl.ds(e0, CE)],
                                 dst_v[b], lsem[b])]

    def gather_chunk(b, ne):
        def gbody(i, carry):
            for u in range(GU):
                o = (i * GU + u) * 16
                idx = src_v[b][pl.ds(o, 16)]
                val_v[b][pl.ds(o, 16)] = plsc.load_gather(x_v, [idx])
            return carry
        lax.fori_loop(0, ne // (16 * GU), gbody, 0)

    def fire_scatter(ic):
        b = ic % NBUF
        return pltpu.async_copy(val_v[b], acc_sh.at[dst_v[b]], ssem[b],
                                add=True)

    loads = {}
    scatters = {}
    loads[0] = load_idx(0)
    loads[1] = load_idx(1)
    plsc.subcore_barrier()
    xcp.wait()
    for ic in range(NCH):
        b = ic % NBUF
        for cp in loads.pop(ic):
            cp.wait()
        gather_chunk(b, CE)
        if ic >= 1:
            scatters.pop(ic - 1).wait()
        if ic + 2 < NCH:
            loads[ic + 2] = load_idx(ic + 2)
        scatters[ic] = fire_scatter(ic)
    scatters.pop(NCH - 1).wait()

    @pl.when(wid < N_EXTRA)
    def _():
        e0 = EXTRA0 + wid * EXTRA_CE
        pltpu.sync_copy(edge_hbm.at[0, pl.ds(e0, EXTRA_CE)],
                        src_v[0].at[pl.ds(0, EXTRA_CE)])
        pltpu.sync_copy(edge_hbm.at[1, pl.ds(e0, EXTRA_CE)],
                        dst_v[0].at[pl.ds(0, EXTRA_CE)])
        gather_chunk(0, EXTRA_CE)
        pltpu.async_copy(val_v[0].at[pl.ds(0, EXTRA_CE)],
                         acc_sh.at[dst_v[0].at[pl.ds(0, EXTRA_CE)]],
                         ssem[0], add=True).wait()

    plsc.subcore_barrier()

    off = 0
    for piece in PIECES:
        pltpu.sync_copy(acc_sh.at[pl.ds(s * SLICE + off, piece)],
                        val_v0.at[pl.ds(0, piece)])
        pltpu.sync_copy(val_v0.at[pl.ds(0, piece)],
                        out_hbm.at[pl.ds(c * ACC_PAD + s * SLICE + off,
                                         piece)])
        off += piece


_sc_fn = functools.partial(
    pl.kernel,
    out_type=jax.ShapeDtypeStruct((NC * ACC_PAD,), jnp.float32),
    mesh=plsc.VectorSubcoreMesh(core_axis_name="c", subcore_axis_name="s"),
    compiler_params=pltpu.CompilerParams(needs_layout_passes=False),
    scratch_types=(
        [pltpu.VMEM((CE,), jnp.int32) for _ in range(3)] +
        [pltpu.VMEM((CE,), jnp.int32) for _ in range(3)] +
        [pltpu.VMEM((CE,), jnp.float32) for _ in range(3)] +
        [pltpu.VMEM((N_NODES,), jnp.float32),
         pltpu.VMEM_SHARED((ACC_PAD,), jnp.float32)] +
        [pltpu.SemaphoreType.DMA for _ in range(7)]
    ),
)(_sc_body)


def _finish_body(w_ref, p_ref, o_ref):
    o_ref[...] = jnp.tanh(w_ref[0] * (p_ref[0] + p_ref[1]))


_finish = pl.pallas_call(
    _finish_body,
    out_shape=jax.ShapeDtypeStruct((ACC_PAD // 128, 128), jnp.float32),
    in_specs=[
        pl.BlockSpec(memory_space=pltpu.SMEM),
        pl.BlockSpec(memory_space=pltpu.VMEM),
    ],
    out_specs=pl.BlockSpec(memory_space=pltpu.VMEM),
)


def kernel(x, edge_index, weight):
    zeros = jnp.zeros((PIECES[0],), jnp.float32)
    partial = _sc_fn(x, edge_index, zeros)
    out2d = _finish(jnp.reshape(weight, (1,)),
                    partial.reshape(NC, ACC_PAD // 128, 128))
    return out2d.reshape(-1)[:N_NODES]


# speedup vs baseline: 1.0268x; 1.0268x over previous
import functools

import jax
import jax.numpy as jnp
from jax import lax
from jax.experimental import pallas as pl
from jax.experimental.pallas import tpu as pltpu
from jax.experimental.pallas import tpu_sc as plsc

N_NODES = 100000
N_EDGES = 6400000

NC, NS = 2, 16
NW = NC * NS
EPW = 199680
CE = 2560
NCH = EPW // CE
NBUF = 3
GU = 4
EXTRA0 = NW * EPW
EXTRA_CE = 1024
N_EXTRA = (N_EDGES - EXTRA0) // EXTRA_CE

SLICE = 6256
ACC_PAD = NS * SLICE
PIECES = (2560, 2560, 1136)


def _sc_body(x_hbm, edge_hbm, zeros_hbm, out_hbm,
             src_v0, src_v1, src_v2, dst_v0, dst_v1, dst_v2,
             val_v0, val_v1, val_v2, x_v, acc_sh,
             lsem0, lsem1, lsem2, ssem0, ssem1, ssem2, xsem):
    src_v = (src_v0, src_v1, src_v2)
    dst_v = (dst_v0, dst_v1, dst_v2)
    val_v = (val_v0, val_v1, val_v2)
    lsem = (lsem0, lsem1, lsem2)
    ssem = (ssem0, ssem1, ssem2)
    c = lax.axis_index("c")
    s = lax.axis_index("s")
    wid = s * NC + c

    xcp = pltpu.async_copy(x_hbm, x_v, xsem)
    off = 0
    for piece in PIECES:
        pltpu.sync_copy(zeros_hbm.at[pl.ds(0, piece)],
                        val_v0.at[pl.ds(0, piece)])
        pltpu.sync_copy(val_v0.at[pl.ds(0, piece)],
                        acc_sh.at[pl.ds(s * SLICE + off, piece)])
        off += piece
    xcp.wait()
    plsc.subcore_barrier()

    base_e = wid * EPW

    def load_idx(ic):
        b = ic % NBUF
        e0 = base_e + ic * CE
        return [pltpu.async_copy(edge_hbm.at[0, pl.ds(e0, CE)], src_v[b],
                                 lsem[b]),
                pltpu.async_copy(edge_hbm.at[1, pl.ds(e0, CE)],
                                 dst_v[b], lsem[b])]

    def gather_chunk(b, ne):
        @plsc.parallel_loop(0, ne // 16, unroll=GU)
        def _(i):
            o = i * 16
            idx = src_v[b][pl.ds(o, 16)]
            val_v[b][pl.ds(o, 16)] = plsc.load_gather(x_v, [idx])

    def fire_scatter(ic):
        b = ic % NBUF
        return pltpu.async_copy(val_v[b], acc_sh.at[dst_v[b]], ssem[b],
                                add=True)

    loads = {}
    scatters = {}
    loads[0] = load_idx(0)
    loads[1] = load_idx(1)
    for ic in range(NCH):
        b = ic % NBUF
        for cp in loads.pop(ic):
            cp.wait()
        gather_chunk(b, CE)
        if ic >= 1:
            scatters.pop(ic - 1).wait()
        if ic + 2 < NCH:
            loads[ic + 2] = load_idx(ic + 2)
        scatters[ic] = fire_scatter(ic)
    scatters.pop(NCH - 1).wait()

    @pl.when(wid < N_EXTRA)
    def _():
        e0 = EXTRA0 + wid * EXTRA_CE
        pltpu.sync_copy(edge_hbm.at[0, pl.ds(e0, EXTRA_CE)],
                        src_v[0].at[pl.ds(0, EXTRA_CE)])
        pltpu.sync_copy(edge_hbm.at[1, pl.ds(e0, EXTRA_CE)],
                        dst_v[0].at[pl.ds(0, EXTRA_CE)])
        gather_chunk(0, EXTRA_CE)
        pltpu.async_copy(val_v[0].at[pl.ds(0, EXTRA_CE)],
                         acc_sh.at[dst_v[0].at[pl.ds(0, EXTRA_CE)]],
                         ssem[0], add=True).wait()

    plsc.subcore_barrier()

    off = 0
    for piece in PIECES:
        pltpu.sync_copy(acc_sh.at[pl.ds(s * SLICE + off, piece)],
                        val_v0.at[pl.ds(0, piece)])
        pltpu.sync_copy(val_v0.at[pl.ds(0, piece)],
                        out_hbm.at[pl.ds(c * ACC_PAD + s * SLICE + off,
                                         piece)])
        off += piece


_sc_fn = functools.partial(
    pl.kernel,
    out_type=jax.ShapeDtypeStruct((NC * ACC_PAD,), jnp.float32),
    mesh=plsc.VectorSubcoreMesh(core_axis_name="c", subcore_axis_name="s"),
    compiler_params=pltpu.CompilerParams(needs_layout_passes=False),
    scratch_types=(
        [pltpu.VMEM((CE,), jnp.int32) for _ in range(3)] +
        [pltpu.VMEM((CE,), jnp.int32) for _ in range(3)] +
        [pltpu.VMEM((CE,), jnp.float32) for _ in range(3)] +
        [pltpu.VMEM((N_NODES,), jnp.float32),
         pltpu.VMEM_SHARED((ACC_PAD,), jnp.float32)] +
        [pltpu.SemaphoreType.DMA for _ in range(7)]
    ),
)(_sc_body)


def _finish_body(w_ref, p_ref, o_ref):
    o_ref[...] = jnp.tanh(w_ref[0] * (p_ref[0] + p_ref[1]))


_finish = pl.pallas_call(
    _finish_body,
    out_shape=jax.ShapeDtypeStruct((ACC_PAD // 128, 128), jnp.float32),
    in_specs=[
        pl.BlockSpec(memory_space=pltpu.SMEM),
        pl.BlockSpec(memory_space=pltpu.VMEM),
    ],
    out_specs=pl.BlockSpec(memory_space=pltpu.VMEM),
)


def kernel(x, edge_index, weight):
    zeros = jnp.zeros((PIECES[0],), jnp.float32)
    partial = _sc_fn(x, edge_index, zeros)
    out2d = _finish(jnp.reshape(weight, (1,)),
                    partial.reshape(NC, ACC_PAD // 128, 128))
    return out2d.reshape(-1)[:N_NODES]


# speedup vs baseline: 1.0498x; 1.0224x over previous
import functools

import jax
import jax.numpy as jnp
from jax import lax
from jax.experimental import pallas as pl
from jax.experimental.pallas import tpu as pltpu
from jax.experimental.pallas import tpu_sc as plsc

N_NODES = 100000
N_EDGES = 6400000

NC, NS = 2, 16
NW = NC * NS
EPW = 199680
CE = 2560
NCH = EPW // CE
NBUF = 3
GU = 4
EXTRA0 = NW * EPW
EXTRA_CE = 1024
N_EXTRA = (N_EDGES - EXTRA0) // EXTRA_CE

SLICE = 6272
ACC_PAD = NS * SLICE
PIECES = (2560, 2560, 1152)


def _sc_body(x_hbm, edge_hbm, zeros_hbm, out_hbm,
             src_v0, src_v1, src_v2, dst_v0, dst_v1, dst_v2,
             val_v0, val_v1, val_v2, x_v, acc_sh,
             lsem0, lsem1, lsem2, ssem0, ssem1, ssem2, xsem):
    src_v = (src_v0, src_v1, src_v2)
    dst_v = (dst_v0, dst_v1, dst_v2)
    val_v = (val_v0, val_v1, val_v2)
    lsem = (lsem0, lsem1, lsem2)
    ssem = (ssem0, ssem1, ssem2)
    c = lax.axis_index("c")
    s = lax.axis_index("s")
    wid = s * NC + c

    xcp = pltpu.async_copy(x_hbm, x_v, xsem)
    off = 0
    for piece in PIECES:
        pltpu.sync_copy(zeros_hbm.at[pl.ds(0, piece)],
                        val_v0.at[pl.ds(0, piece)])
        pltpu.sync_copy(val_v0.at[pl.ds(0, piece)],
                        acc_sh.at[pl.ds(s * SLICE + off, piece)])
        off += piece

    base_e = wid * EPW

    def load_idx(ic):
        b = ic % NBUF
        e0 = base_e + ic * CE
        return [pltpu.async_copy(edge_hbm.at[0, pl.ds(e0, CE)], src_v[b],
                                 lsem[b]),
                pltpu.async_copy(edge_hbm.at[1, pl.ds(e0, CE)],
                                 dst_v[b], lsem[b])]

    def gather_chunk(b, ne):
        @plsc.parallel_loop(0, ne // 16, unroll=GU)
        def _(i):
            o = i * 16
            idx = src_v[b][pl.ds(o, 16)]
            val_v[b][pl.ds(o, 16)] = plsc.load_gather(x_v, [idx])

    def fire_scatter(ic):
        b = ic % NBUF
        return pltpu.async_copy(val_v[b], acc_sh.at[dst_v[b]], ssem[b],
                                add=True)

    loads = {}
    scatters = {}
    loads[0] = load_idx(0)
    loads[1] = load_idx(1)
    plsc.subcore_barrier()
    xcp.wait()
    for ic in range(NCH):
        b = ic % NBUF
        for cp in loads.pop(ic):
            cp.wait()
        gather_chunk(b, CE)
        if ic >= 1:
            scatters.pop(ic - 1).wait()
        if ic + 2 < NCH:
            loads[ic + 2] = load_idx(ic + 2)
        scatters[ic] = fire_scatter(ic)
    scatters.pop(NCH - 1).wait()

    @pl.when(wid < N_EXTRA)
    def _():
        e0 = EXTRA0 + wid * EXTRA_CE
        pltpu.sync_copy(edge_hbm.at[0, pl.ds(e0, EXTRA_CE)],
                        src_v[0].at[pl.ds(0, EXTRA_CE)])
        pltpu.sync_copy(edge_hbm.at[1, pl.ds(e0, EXTRA_CE)],
                        dst_v[0].at[pl.ds(0, EXTRA_CE)])
        gather_chunk(0, EXTRA_CE)
        pltpu.async_copy(val_v[0].at[pl.ds(0, EXTRA_CE)],
                         acc_sh.at[dst_v[0].at[pl.ds(0, EXTRA_CE)]],
                         ssem[0], add=True).wait()

    plsc.subcore_barrier()

    off = 0
    for piece in PIECES:
        pltpu.sync_copy(acc_sh.at[pl.ds(s * SLICE + off, piece)],
                        val_v0.at[pl.ds(0, piece)])
        pltpu.sync_copy(val_v0.at[pl.ds(0, piece)],
                        out_hbm.at[pl.ds(c * ACC_PAD + s * SLICE + off,
                                         piece)])
        off += piece


_sc_fn = functools.partial(
    pl.kernel,
    out_type=jax.ShapeDtypeStruct((NC * ACC_PAD,), jnp.float32),
    mesh=plsc.VectorSubcoreMesh(core_axis_name="c", subcore_axis_name="s"),
    compiler_params=pltpu.CompilerParams(needs_layout_passes=False),
    scratch_types=(
        [pltpu.VMEM((CE,), jnp.int32) for _ in range(3)] +
        [pltpu.VMEM((CE,), jnp.int32) for _ in range(3)] +
        [pltpu.VMEM((CE,), jnp.float32) for _ in range(3)] +
        [pltpu.VMEM((N_NODES,), jnp.float32),
         pltpu.VMEM_SHARED((ACC_PAD,), jnp.float32)] +
        [pltpu.SemaphoreType.DMA for _ in range(7)]
    ),
)(_sc_body)


def _finish_body(w_ref, p_ref, o_ref):
    o_ref[...] = jnp.tanh(w_ref[0] * (p_ref[0] + p_ref[1]))


_finish = pl.pallas_call(
    _finish_body,
    out_shape=jax.ShapeDtypeStruct((ACC_PAD // 128, 128), jnp.float32),
    in_specs=[
        pl.BlockSpec(memory_space=pltpu.SMEM),
        pl.BlockSpec(memory_space=pltpu.VMEM),
    ],
    out_specs=pl.BlockSpec(memory_space=pltpu.VMEM),
)


def kernel(x, edge_index, weight):
    zeros = jnp.zeros((PIECES[0],), jnp.float32)
    partial = _sc_fn(x, edge_index, zeros)
    out2d = _finish(jnp.reshape(weight, (1,)),
                    partial.reshape(NC, ACC_PAD // 128, 128))
    return out2d.reshape(-1)[:N_NODES]
